# baseline (speedup 1.0000x reference)
import functools

import jax
import jax.numpy as jnp
from jax import lax
from jax.experimental import pallas as pl
from jax.experimental.pallas import tpu as pltpu
from jax.experimental.pallas import tpu_sc as plsc

_COLS_PER_BLOCK = 2048
_B = 4.0


def _block_body(logits_col_ref, logits_row_ref, prior_row_ref, g_ref,
                d_ref, lp_ref, ln_ref, cw_ref, row_ref, sc_ref):
    k = logits_col_ref.shape[0]
    c = g_ref.shape[1]

    @pl.when(pl.program_id(0) == 0)
    def _prologue():
        lr = logits_row_ref[...]
        pr = prior_row_ref[...]
        m_l = jnp.max(lr, axis=1, keepdims=True)
        lse_l = m_l + jnp.log(jnp.sum(jnp.exp(lr - m_l), axis=1, keepdims=True))
        ln_row = lr - lse_l
        ln_ref[...] = logits_col_ref[...] - lse_l
        cw_ref[0:1, :] = jnp.ones((1, k), jnp.float32)
        cw_ref[1:2, :] = jnp.exp(pr - ln_row)
        sc_ref[0] = jnp.sum(ln_row - pr)
        row_ref[...] = lax.broadcasted_iota(jnp.int32, (k, 1), 0).astype(jnp.float32)

    ln = ln_ref[...]
    s_const = sc_ref[0]

    g = g_ref[...]

    e1 = jnp.exp((-_B) - g)
    sums = jnp.dot(cw_ref[...], e1, preferred_element_type=jnp.float32)
    s1 = sums[0:1, :]
    s2 = sums[1:2, :]
    lp_ref[...] = s_const + float(k) * jnp.log(s2 / s1)

    z = ln + g
    mz = jnp.max(z, axis=0, keepdims=True)
    row = row_ref[...]
    big = jnp.float32(2.0**30)
    idx = jnp.min(jnp.where(z == mz, row, big), axis=0, keepdims=True)
    d_ref[...] = (row == idx).astype(jnp.float32)


_ZN = 2 * 1024 * 1024
_NW = 32
_PW = _ZN // _NW
_CH = 2048


def _sc_probe():
    mesh = plsc.VectorSubcoreMesh(core_axis_name="c", subcore_axis_name="s")

    @functools.partial(
        pl.kernel, mesh=mesh,
        out_type=jax.ShapeDtypeStruct((_ZN,), jnp.float32),
        scratch_types=[pltpu.VMEM((_CH,), jnp.float32)],
    )
    def probe(out_hbm, buf):
        cid = lax.axis_index("c")
        sid = lax.axis_index("s")
        wid = sid * 2 + cid

        def fill(i, carry):
            buf[pl.ds(i * 16, 16)] = jnp.zeros((16,), jnp.float32)
            return carry

        lax.fori_loop(0, _CH // 16, fill, 0)
        base = wid * _PW

        def cp(j, carry):
            pltpu.sync_copy(buf, out_hbm.at[pl.ds(base + j * _CH, _CH)])
            return carry

        lax.fori_loop(0, _PW // _CH, cp, 0)

    return probe()


def kernel(num_samples, temperature, logits, prior_logits, gumbel):
    del num_samples, temperature
    n, k = gumbel.shape
    c = _COLS_PER_BLOCK
    gt = gumbel.T
    dt, lp = pl.pallas_call(
        _block_body,
        grid=(n // c,),
        compiler_params=pltpu.CompilerParams(
            dimension_semantics=("arbitrary",),
        ),
        in_specs=[
            pl.BlockSpec((k, 1), lambda i: (0, 0)),
            pl.BlockSpec((1, k), lambda i: (0, 0)),
            pl.BlockSpec((1, k), lambda i: (0, 0)),
            pl.BlockSpec((k, c), lambda i: (0, i)),
        ],
        out_specs=[
            pl.BlockSpec((k, c), lambda i: (0, i)),
            pl.BlockSpec((1, c), lambda i: (0, i)),
        ],
        out_shape=[
            jax.ShapeDtypeStruct((k, n), jnp.float32),
            jax.ShapeDtypeStruct((1, n), jnp.float32),
        ],
        scratch_shapes=[
            pltpu.VMEM((k, 1), jnp.float32),
            pltpu.VMEM((2, k), jnp.float32),
            pltpu.VMEM((k, 1), jnp.float32),
            pltpu.SMEM((1,), jnp.float32),
        ],
    )(logits.reshape(k, 1), logits.reshape(1, k), prior_logits.reshape(1, k), gt)
    z = _sc_probe()
    lp = lp + z[0]
    return dt.T, lp.reshape(n)


# speedup vs baseline: 1.4536x; 1.4536x over previous
import jax
import jax.numpy as jnp
from jax import lax
from jax.experimental import pallas as pl
from jax.experimental.pallas import tpu as pltpu

_COLS_PER_BLOCK = 2048
_B = 4.0


def _block_body(logits_row_ref, prior_row_ref, g_ref,
                d_ref, lp_ref, ln_ref, cw_ref, row_ref, sc_ref):
    k = logits_row_ref.shape[1]
    c = g_ref.shape[1]

    @pl.when(pl.program_id(0) == 0)
    def _prologue():
        lr = logits_row_ref[...]
        pr = prior_row_ref[...]
        m_l = jnp.max(lr, axis=1, keepdims=True)
        lse_l = m_l + jnp.log(jnp.sum(jnp.exp(lr - m_l), axis=1, keepdims=True))
        ln_row = lr - lse_l
        ln_ref[...] = jnp.transpose(ln_row, (1, 0))
        cw_ref[0:1, :] = jnp.ones((1, k), jnp.float32)
        cw_ref[1:2, :] = jnp.exp(pr - ln_row)
        sc_ref[0] = jnp.sum(ln_row - pr)
        row_ref[...] = lax.broadcasted_iota(jnp.int32, (k, 1), 0).astype(jnp.float32)

    ln = ln_ref[...]
    s_const = sc_ref[0]

    g = g_ref[...]

    e1 = jnp.exp((-_B) - g)
    sums = jnp.dot(cw_ref[...], e1, preferred_element_type=jnp.float32)
    s1 = sums[0:1, :]
    s2 = sums[1:2, :]
    lp_ref[...] = s_const + float(k) * jnp.log(s2 / s1)

    z = ln + g
    mz = jnp.max(z, axis=0, keepdims=True)
    row = row_ref[...]
    big = jnp.float32(2.0**30)
    idx = jnp.min(jnp.where(z == mz, row, big), axis=0, keepdims=True)
    d_ref[...] = (row == idx).astype(jnp.float32)


def kernel(num_samples, temperature, logits, prior_logits, gumbel):
    del num_samples, temperature
    n, k = gumbel.shape
    c = _COLS_PER_BLOCK
    gt = gumbel.T
    dt, lp = pl.pallas_call(
        _block_body,
        grid=(n // c,),
        compiler_params=pltpu.CompilerParams(
            dimension_semantics=("arbitrary",),
        ),
        in_specs=[
            pl.BlockSpec((1, k), lambda i: (0, 0)),
            pl.BlockSpec((1, k), lambda i: (0, 0)),
            pl.BlockSpec((k, c), lambda i: (0, i)),
        ],
        out_specs=[
            pl.BlockSpec((k, c), lambda i: (0, i)),
            pl.BlockSpec((1, c), lambda i: (0, i)),
        ],
        out_shape=[
            jax.ShapeDtypeStruct((k, n), jnp.float32),
            jax.ShapeDtypeStruct((1, n), jnp.float32),
        ],
        scratch_shapes=[
            pltpu.VMEM((k, 1), jnp.float32),
            pltpu.VMEM((2, k), jnp.float32),
            pltpu.VMEM((k, 1), jnp.float32),
            pltpu.SMEM((1,), jnp.float32),
        ],
    )(logits.reshape(1, k), prior_logits.reshape(1, k), gt)
    return dt.T, lp.reshape(n)
